# Initial kernel scaffold; baseline (speedup 1.0000x reference)
import jax
import jax.numpy as jnp
from jax.experimental import pallas as pl


def kernel(x, edge_index, W, b, gamma, beta):
    raise NotImplementedError("write your pallas kernel here")


# speedup vs baseline: 20.0509x; 20.0509x over previous
import functools

import jax
import jax.numpy as jnp
from jax import lax
from jax.experimental import pallas as pl
from jax.experimental.pallas import tpu as pltpu
from jax.experimental.pallas import tpu_sc as plsc

N = 10000
E = 320000
D = 128
EPS = 1e-5
NEG_SLOPE = 0.01

NC, NS = 2, 16
NW = NC * NS
EPT = E // NW
CH = 128
NCH = EPT // CH
TAIL = EPT - NCH * CH

BM = 400
GRID = N // BM
NP = 10240

_mesh = plsc.VectorSubcoreMesh(
    core_axis_name="c", subcore_axis_name="s", num_cores=NC, num_subcores=NS)


@functools.partial(
    pl.kernel,
    out_type=jax.ShapeDtypeStruct((NC * N,), jnp.float32),
    mesh=_mesh,
    scratch_types=[
        pltpu.VMEM_SHARED((N,), jnp.float32),
        pltpu.VMEM((CH,), jnp.int32),
        pltpu.VMEM((TAIL,), jnp.int32),
        pltpu.VMEM((CH,), jnp.float32),
        pltpu.VMEM((TAIL,), jnp.float32),
        pltpu.VMEM((2000,), jnp.float32),
    ],
)
def _deg_kernel(dst_hbm, out_hbm, acc, didx, didx_t, ones, ones_t, zbuf):
    c = lax.axis_index("c")
    s = lax.axis_index("s")
    wid = s * NC + c

    one16 = jnp.full((16,), 1.0, dtype=jnp.float32)
    zero16 = jnp.zeros((16,), dtype=jnp.float32)

    @pl.loop(0, CH // 16)
    def _(i):
        ones[pl.ds(i * 16, 16)] = one16
    ones_t[...] = one16

    @pl.when(s == 0)
    def _():
        @pl.loop(0, 2000 // 16)
        def _(i):
            zbuf[pl.ds(i * 16, 16)] = zero16
        for k in range(5):
            pltpu.sync_copy(zbuf, acc.at[pl.ds(k * 2000, 2000)])

    plsc.subcore_barrier()

    ebase = wid * EPT

    @pl.loop(0, NCH)
    def _(k):
        pltpu.sync_copy(dst_hbm.at[pl.ds(ebase + k * CH, CH)], didx)
        pltpu.sync_copy(ones, acc.at[didx], add=True)

    pltpu.sync_copy(dst_hbm.at[pl.ds(ebase + NCH * CH, TAIL)], didx_t)
    pltpu.sync_copy(ones_t, acc.at[didx_t], add=True)

    plsc.subcore_barrier()

    @pl.when(s == 0)
    def _():
        for k in range(5):
            pltpu.sync_copy(acc.at[pl.ds(k * 2000, 2000)], zbuf)
            pltpu.sync_copy(zbuf, out_hbm.at[pl.ds(c * N + k * 2000, 2000)])


def _hs_body(deg_ref, x_ref, w_ref, hs_ref, dinv_ref):
    deg = 1.0 + deg_ref[0] + deg_ref[1]
    dinv = lax.rsqrt(deg)
    h = jnp.dot(x_ref[...], w_ref[...], preferred_element_type=jnp.float32)
    hs_ref[...] = h * dinv
    dinv_ref[...] = dinv


_hs_call = pl.pallas_call(
    _hs_body,
    grid=(GRID,),
    in_specs=[
        pl.BlockSpec((NC, BM, 1), lambda i: (0, i, 0)),
        pl.BlockSpec((BM, D), lambda i: (i, 0)),
        pl.BlockSpec((D, D), lambda i: (0, 0)),
    ],
    out_specs=[
        pl.BlockSpec((BM, D), lambda i: (i, 0)),
        pl.BlockSpec((BM, 1), lambda i: (i, 0)),
    ],
    out_shape=[
        jax.ShapeDtypeStruct((N, D), jnp.float32),
        jax.ShapeDtypeStruct((N, 1), jnp.float32),
    ],
)


@functools.partial(
    pl.kernel,
    out_type=jax.ShapeDtypeStruct((NC * NP, D), jnp.float32),
    mesh=_mesh,
    scratch_types=[
        pltpu.VMEM_SHARED((NP, D), jnp.float32),
        pltpu.VMEM((CH, D), jnp.float32),
        pltpu.VMEM((CH,), jnp.int32),
        pltpu.VMEM((CH,), jnp.int32),
        pltpu.VMEM((TAIL, D), jnp.float32),
        pltpu.VMEM((TAIL,), jnp.int32),
        pltpu.VMEM((TAIL,), jnp.int32),
        pltpu.SemaphoreType.DMA,
    ],
)
def _agg_kernel(src_hbm, dst_hbm, hs_hbm, out_hbm, acc, rows, sidx, didx,
                rows_t, sidx_t, didx_t, sem):
    c = lax.axis_index("c")
    s = lax.axis_index("s")
    wid = s * NC + c

    zero16 = jnp.zeros((16,), dtype=jnp.float32)

    @pl.loop(0, CH)
    def _(r):
        @pl.loop(0, D // 16)
        def _(j):
            rows[r, pl.ds(j * 16, 16)] = zero16

    rbase = s * (NP // NS)
    for k in range(5):
        pltpu.sync_copy(rows, acc.at[pl.ds(rbase + k * CH, CH)])

    plsc.subcore_barrier()

    ebase = wid * EPT

    @pl.loop(0, NCH)
    def _(k):
        base = ebase + k * CH
        pltpu.sync_copy(src_hbm.at[pl.ds(base, CH)], sidx)
        pltpu.sync_copy(dst_hbm.at[pl.ds(base, CH)], didx)
        pltpu.async_copy(hs_hbm.at[sidx], rows, sem).wait()
        pltpu.sync_copy(rows, acc.at[didx], add=True)

    tbase = ebase + NCH * CH
    pltpu.sync_copy(src_hbm.at[pl.ds(tbase, TAIL)], sidx_t)
    pltpu.sync_copy(dst_hbm.at[pl.ds(tbase, TAIL)], didx_t)
    pltpu.async_copy(hs_hbm.at[sidx_t], rows_t, sem).wait()
    pltpu.sync_copy(rows_t, acc.at[didx_t], add=True)

    plsc.subcore_barrier()

    for k in range(5):
        pltpu.sync_copy(acc.at[pl.ds(rbase + k * CH, CH)], rows)
        pltpu.sync_copy(rows, out_hbm.at[pl.ds(c * NP + rbase + k * CH, CH)])


def _zstats_body(y_ref, hs_ref, dinv_ref, b_ref, z_ref, sum_ref, sq_ref,
                 acc_s, acc_q):
    i = pl.program_id(0)
    t = (y_ref[0] + y_ref[1] + hs_ref[...]) * dinv_ref[...] + b_ref[...]
    z = jnp.where(t >= 0, t, NEG_SLOPE * t)
    z_ref[...] = z

    @pl.when(i == 0)
    def _():
        acc_s[...] = jnp.zeros_like(acc_s)
        acc_q[...] = jnp.zeros_like(acc_q)

    acc_s[...] += jnp.sum(z, axis=0, keepdims=True)
    acc_q[...] += jnp.sum(z * z, axis=0, keepdims=True)

    @pl.when(i == pl.num_programs(0) - 1)
    def _():
        sum_ref[...] = acc_s[...]
        sq_ref[...] = acc_q[...]


_zstats_call = pl.pallas_call(
    _zstats_body,
    grid=(GRID,),
    in_specs=[
        pl.BlockSpec((NC, BM, D), lambda i: (0, i, 0)),
        pl.BlockSpec((BM, D), lambda i: (i, 0)),
        pl.BlockSpec((BM, 1), lambda i: (i, 0)),
        pl.BlockSpec((1, D), lambda i: (0, 0)),
    ],
    out_specs=[
        pl.BlockSpec((BM, D), lambda i: (i, 0)),
        pl.BlockSpec((1, D), lambda i: (0, 0)),
        pl.BlockSpec((1, D), lambda i: (0, 0)),
    ],
    out_shape=[
        jax.ShapeDtypeStruct((N, D), jnp.float32),
        jax.ShapeDtypeStruct((1, D), jnp.float32),
        jax.ShapeDtypeStruct((1, D), jnp.float32),
    ],
    scratch_shapes=[
        pltpu.VMEM((1, D), jnp.float32),
        pltpu.VMEM((1, D), jnp.float32),
    ],
)


def _bn_body(z_ref, sum_ref, sq_ref, gamma_ref, beta_ref, out_ref,
             scale_s, shift_s):
    i = pl.program_id(0)

    @pl.when(i == 0)
    def _():
        mean = sum_ref[...] * (1.0 / N)
        var = sq_ref[...] * (1.0 / N) - mean * mean
        rstd = lax.rsqrt(var + EPS)
        scale_s[...] = gamma_ref[...] * rstd
        shift_s[...] = beta_ref[...] - mean * gamma_ref[...] * rstd

    out_ref[...] = z_ref[...] * scale_s[...] + shift_s[...]


_bn_call = pl.pallas_call(
    _bn_body,
    grid=(GRID,),
    in_specs=[
        pl.BlockSpec((BM, D), lambda i: (i, 0)),
        pl.BlockSpec((1, D), lambda i: (0, 0)),
        pl.BlockSpec((1, D), lambda i: (0, 0)),
        pl.BlockSpec((1, D), lambda i: (0, 0)),
        pl.BlockSpec((1, D), lambda i: (0, 0)),
    ],
    out_specs=pl.BlockSpec((BM, D), lambda i: (i, 0)),
    out_shape=jax.ShapeDtypeStruct((N, D), jnp.float32),
    scratch_shapes=[
        pltpu.VMEM((1, D), jnp.float32),
        pltpu.VMEM((1, D), jnp.float32),
    ],
)


def kernel(x, edge_index, W, b, gamma, beta):
    src = edge_index[0]
    dst = edge_index[1]

    degp = _deg_kernel(dst).reshape(NC, N, 1)
    hs, dinv = _hs_call(degp, x, W)
    y = _agg_kernel(src, dst, hs).reshape(NC, NP, D)
    z, s1, s2 = _zstats_call(y, hs, dinv, b.reshape(1, D))
    return _bn_call(z, s1, s2, gamma.reshape(1, D), beta.reshape(1, D))
